# baseline (speedup 1.0000x reference)
import functools

import jax
import jax.numpy as jnp
from jax import lax
from jax.experimental import pallas as pl
from jax.experimental.pallas import tpu as pltpu
from jax.experimental.pallas import tpu_sc as plsc

_N = 10000
_E = 320000
_DIN = 128
_DH = 64
_DOUT = 128

_NC = 2
_NS = 16
_NW = _NC * _NS

_CHUNK = 128
_NB = 8
_LA = 4
_K = 80
_EPAD = _NW * _K * _CHUNK
_NACC = 10240
_RPT = _NACC // _NS
_ZR = 128


def _matmul_body(x_ref, w_ref, o_ref):
    o_ref[...] = jnp.dot(x_ref[...], w_ref[...],
                         preferred_element_type=jnp.float32)


def _mid_body(p_ref, parts_ref, b1_ref, w2_ref, b2_ref, w3_ref, o_ref):
    z = jnp.maximum(
        p_ref[...] + parts_ref[0, 0:_N, :] + parts_ref[1, 0:_N, :]
        + b1_ref[...], 0.0)
    h = jnp.maximum(
        jnp.dot(z, w2_ref[...], preferred_element_type=jnp.float32)
        + b2_ref[...], 0.0)
    o_ref[...] = jnp.dot(h, w3_ref[...], preferred_element_type=jnp.float32)


def _out_body(p_ref, parts_ref, b3_ref, w4_ref, b4_ref, o_ref):
    z = jnp.maximum(
        p_ref[...] + parts_ref[0, 0:_N, :] + parts_ref[1, 0:_N, :]
        + b3_ref[...], 0.0)
    o = jnp.dot(z, w4_ref[...], preferred_element_type=jnp.float32) + b4_ref[...]
    s = o - jnp.max(o, axis=1, keepdims=True)
    o_ref[...] = s - jnp.log(jnp.sum(jnp.exp(s), axis=1, keepdims=True))


_matmul = pl.pallas_call(
    _matmul_body,
    out_shape=jax.ShapeDtypeStruct((_N, _DH), jnp.float32),
)

_mid = pl.pallas_call(
    _mid_body,
    out_shape=jax.ShapeDtypeStruct((_N, _DH), jnp.float32),
)

_out = pl.pallas_call(
    _out_body,
    out_shape=jax.ShapeDtypeStruct((_N, _DOUT), jnp.float32),
)


def _make_segsum():
    mesh = plsc.VectorSubcoreMesh(core_axis_name="c", subcore_axis_name="s")

    @functools.partial(
        pl.kernel,
        mesh=mesh,
        compiler_params=pltpu.CompilerParams(use_tc_tiling_on_sc=False),
        out_type=jax.ShapeDtypeStruct((_NC, _NACC, _DH), jnp.float32),
        scratch_types=[
            pltpu.VMEM((_K, _CHUNK), jnp.int32),
            pltpu.VMEM((_K, _CHUNK), jnp.int32),
            pltpu.VMEM((_NB, _CHUNK, _DH), jnp.float32),
            pltpu.VMEM_SHARED((_NACC, _DH), jnp.float32),
        ] + [pltpu.SemaphoreType.DMA] * (2 * _NB),
    )
    def segsum(p_hbm, src_hbm, dst_hbm, out_hbm,
               src_v, dst_v, rows_v, acc, *sems):
        cid = lax.axis_index("c")
        sid = lax.axis_index("s")
        wid = cid * _NS + sid
        sem_g = sems[:_NB]
        sem_s = sems[_NB:]

        pltpu.sync_copy(src_hbm.at[wid], src_v)
        pltpu.sync_copy(dst_hbm.at[wid], dst_v)

        def zrow(r, carry):
            for c in range(_DH // 16):
                rows_v[0, r, pl.ds(c * 16, 16)] = jnp.zeros((16,),
                                                            jnp.float32)
            return carry
        lax.fori_loop(0, _ZR, zrow, 0)
        base = sid * _RPT
        for t in range(_RPT // _ZR):
            pltpu.sync_copy(rows_v.at[0], acc.at[pl.ds(base + t * _ZR, _ZR)])
        plsc.subcore_barrier()

        def start_gather(j, b):
            pltpu.async_copy(p_hbm.at[src_v.at[j]], rows_v.at[b], sem_g[b])

        def wait_gather(j, b):
            pltpu.make_async_copy(
                p_hbm.at[src_v.at[j]], rows_v.at[b], sem_g[b]).wait()

        def start_scatter(j, b):
            pltpu.async_copy(rows_v.at[b], acc.at[dst_v.at[j]], sem_s[b],
                             add=True)

        def wait_scatter(j, b):
            pltpu.make_async_copy(
                rows_v.at[b], acc.at[dst_v.at[j]], sem_s[b]).wait()

        for u in range(_LA):
            start_gather(u, u)
        for u in range(_LA):
            wait_gather(u, u)
            start_scatter(u, u)
            start_gather(u + _LA, (u + _LA) % _NB)

        def body(it, carry):
            j0 = _LA + 8 * it
            for u in range(8):
                j = j0 + u
                b = (_LA + u) % _NB
                wait_gather(j, b)
                start_scatter(j, b)
                wait_scatter(j - _LA, u % _NB)
                start_gather(j + _LA, u % _NB)
            return carry

        lax.fori_loop(0, (_K - 2 * _LA) // 8, body, 0)

        for u in range(_LA):
            j = _K - _LA + u
            wait_gather(j, (_LA + u) % _NB)
            start_scatter(j, (_LA + u) % _NB)
            wait_scatter(j - _LA, u % _NB)
        for u in range(_LA):
            wait_scatter(_K - _LA + u, (_LA + u) % _NB)
        plsc.subcore_barrier()

        pltpu.sync_copy(acc.at[pl.ds(base, _RPT)],
                        out_hbm.at[cid, pl.ds(base, _RPT)])

    return segsum


_segsum = _make_segsum()


def kernel(x, edge_index, W1, b1, W2, b2, W3, b3, W4, b4):
    src = edge_index[0].astype(jnp.int32)
    dst = edge_index[1].astype(jnp.int32)
    npad = _EPAD - _E
    pad_src = jnp.arange(npad, dtype=jnp.int32) % _N
    pad_dst = _N + (jnp.arange(npad, dtype=jnp.int32) % (_NACC - _N))
    src_t = jnp.concatenate([src, pad_src]).reshape(_NW, _K, _CHUNK)
    dst_t = jnp.concatenate([dst, pad_dst]).reshape(_NW, _K, _CHUNK)

    b1r = b1.reshape(1, _DH)
    b2r = b2.reshape(1, _DH)
    b3r = b3.reshape(1, _DH)
    b4r = b4.reshape(1, _DOUT)

    p1 = _matmul(x, W1)
    parts1 = _segsum(p1, src_t, dst_t)
    p2 = _mid(p1, parts1, b1r, W2, b2r, W3)
    parts2 = _segsum(p2, src_t, dst_t)
    return _out(p2, parts2, b3r, W4, b4r)


# speedup vs baseline: 1.2859x; 1.2859x over previous
import functools

import jax
import jax.numpy as jnp
from jax import lax
from jax.experimental import pallas as pl
from jax.experimental.pallas import tpu as pltpu
from jax.experimental.pallas import tpu_sc as plsc

_N = 10000
_E = 320000
_DIN = 128
_DH = 64
_DOUT = 128

_NC = 2
_NS = 16
_NW = _NC * _NS

_CHUNK = 128
_NB = 8
_LA = 4
_K = 80
_EPAD = _NW * _K * _CHUNK
_NACC = 10240
_RPT = _NACC // _NS
_ZR = 128


def _matmul_body(x_ref, w_ref, o_ref):
    o_ref[...] = jnp.dot(x_ref[...], w_ref[...],
                         preferred_element_type=jnp.float32
                         ).astype(jnp.bfloat16)


def _mid_body(p_ref, parts_ref, b1_ref, w2_ref, b2_ref, w3_ref, o_ref):
    z = jnp.maximum(
        p_ref[...].astype(jnp.float32)
        + parts_ref[0, 0:_N, :].astype(jnp.float32)
        + parts_ref[1, 0:_N, :].astype(jnp.float32)
        + b1_ref[...], 0.0)
    h = jnp.maximum(
        jnp.dot(z, w2_ref[...], preferred_element_type=jnp.float32)
        + b2_ref[...], 0.0)
    o_ref[...] = jnp.dot(h, w3_ref[...], preferred_element_type=jnp.float32
                         ).astype(jnp.bfloat16)


def _out_body(p_ref, parts_ref, b3_ref, w4_ref, b4_ref, o_ref):
    z = jnp.maximum(
        p_ref[...].astype(jnp.float32)
        + parts_ref[0, 0:_N, :].astype(jnp.float32)
        + parts_ref[1, 0:_N, :].astype(jnp.float32)
        + b3_ref[...], 0.0)
    o = jnp.dot(z, w4_ref[...], preferred_element_type=jnp.float32) + b4_ref[...]
    s = o - jnp.max(o, axis=1, keepdims=True)
    o_ref[...] = s - jnp.log(jnp.sum(jnp.exp(s), axis=1, keepdims=True))


_matmul = pl.pallas_call(
    _matmul_body,
    out_shape=jax.ShapeDtypeStruct((_N, _DH), jnp.bfloat16),
)

_mid = pl.pallas_call(
    _mid_body,
    out_shape=jax.ShapeDtypeStruct((_N, _DH), jnp.bfloat16),
)

_out = pl.pallas_call(
    _out_body,
    out_shape=jax.ShapeDtypeStruct((_N, _DOUT), jnp.float32),
)


def _make_segsum():
    mesh = plsc.VectorSubcoreMesh(core_axis_name="c", subcore_axis_name="s")

    @functools.partial(
        pl.kernel,
        mesh=mesh,
        compiler_params=pltpu.CompilerParams(use_tc_tiling_on_sc=False),
        out_type=jax.ShapeDtypeStruct((_NC, _NACC, _DH), jnp.bfloat16),
        scratch_types=[
            pltpu.VMEM((_K, _CHUNK), jnp.int32),
            pltpu.VMEM((_K, _CHUNK), jnp.int32),
            pltpu.VMEM((_NB, _CHUNK, _DH), jnp.bfloat16),
            pltpu.VMEM_SHARED((_NACC, _DH), jnp.bfloat16),
        ] + [pltpu.SemaphoreType.DMA] * (2 * _NB),
    )
    def segsum(p_hbm, src_hbm, dst_hbm, out_hbm,
               src_v, dst_v, rows_v, acc, *sems):
        cid = lax.axis_index("c")
        sid = lax.axis_index("s")
        wid = cid * _NS + sid
        sem_g = sems[:_NB]
        sem_s = sems[_NB:]

        pltpu.sync_copy(src_hbm.at[wid], src_v)
        pltpu.sync_copy(dst_hbm.at[wid], dst_v)

        def zrow(r, carry):
            for c in range(_DH // 32):
                rows_v[0, r, pl.ds(c * 32, 32)] = jnp.zeros((32,),
                                                            jnp.bfloat16)
            return carry
        lax.fori_loop(0, _ZR, zrow, 0)
        base = sid * _RPT
        for t in range(_RPT // _ZR):
            pltpu.sync_copy(rows_v.at[0], acc.at[pl.ds(base + t * _ZR, _ZR)])
        plsc.subcore_barrier()

        def start_gather(j, b):
            pltpu.async_copy(p_hbm.at[src_v.at[j]], rows_v.at[b], sem_g[b])

        def wait_gather(j, b):
            pltpu.make_async_copy(
                p_hbm.at[src_v.at[j]], rows_v.at[b], sem_g[b]).wait()

        def start_scatter(j, b):
            pltpu.async_copy(rows_v.at[b], acc.at[dst_v.at[j]], sem_s[b],
                             add=True)

        def wait_scatter(j, b):
            pltpu.make_async_copy(
                rows_v.at[b], acc.at[dst_v.at[j]], sem_s[b]).wait()

        for u in range(_LA):
            start_gather(u, u)
        for u in range(_LA):
            wait_gather(u, u)
            start_scatter(u, u)
            start_gather(u + _LA, (u + _LA) % _NB)

        def body(it, carry):
            j0 = _LA + 8 * it
            for u in range(8):
                j = j0 + u
                b = (_LA + u) % _NB
                wait_gather(j, b)
                start_scatter(j, b)
                wait_scatter(j - _LA, u % _NB)
                start_gather(j + _LA, u % _NB)
            return carry

        lax.fori_loop(0, (_K - 2 * _LA) // 8, body, 0)

        for u in range(_LA):
            j = _K - _LA + u
            wait_gather(j, (_LA + u) % _NB)
            start_scatter(j, (_LA + u) % _NB)
            wait_scatter(j - _LA, u % _NB)
        for u in range(_LA):
            wait_scatter(_K - _LA + u, (_LA + u) % _NB)
        plsc.subcore_barrier()

        pltpu.sync_copy(acc.at[pl.ds(base, _RPT)],
                        out_hbm.at[cid, pl.ds(base, _RPT)])

    return segsum


_segsum = _make_segsum()


def kernel(x, edge_index, W1, b1, W2, b2, W3, b3, W4, b4):
    src = edge_index[0].astype(jnp.int32)
    dst = edge_index[1].astype(jnp.int32)
    npad = _EPAD - _E
    pad_src = jnp.arange(npad, dtype=jnp.int32) % _N
    pad_dst = _N + (jnp.arange(npad, dtype=jnp.int32) % (_NACC - _N))
    src_t = jnp.concatenate([src, pad_src]).reshape(_NW, _K, _CHUNK)
    dst_t = jnp.concatenate([dst, pad_dst]).reshape(_NW, _K, _CHUNK)

    b1r = b1.reshape(1, _DH)
    b2r = b2.reshape(1, _DH)
    b3r = b3.reshape(1, _DH)
    b4r = b4.reshape(1, _DOUT)

    p1 = _matmul(x, W1)
    parts1 = _segsum(p1, src_t, dst_t)
    p2 = _mid(p1, parts1, b1r, W2, b2r, W3)
    parts2 = _segsum(p2, src_t, dst_t)
    return _out(p2, parts2, b3r, W4, b4r)


# speedup vs baseline: 1.3126x; 1.0207x over previous
import functools

import jax
import jax.numpy as jnp
from jax import lax
from jax.experimental import pallas as pl
from jax.experimental.pallas import tpu as pltpu
from jax.experimental.pallas import tpu_sc as plsc

_N = 10000
_E = 320000
_DIN = 128
_DH = 64
_DOUT = 128

_NC = 2
_NS = 16
_NW = _NC * _NS

_CHUNK = 128
_NB = 8
_LA = 4
_K = 80
_EPAD = _NW * _K * _CHUNK
_NACC = 10240
_RPT = _NACC // _NS
_ZR = 128


def _matmul_body(x_ref, w_ref, o_ref):
    o_ref[...] = jnp.dot(x_ref[...], w_ref[...],
                         preferred_element_type=jnp.float32
                         ).astype(jnp.bfloat16)


def _mid_body(p_ref, parts_ref, b1_ref, w2_ref, b2_ref, w3_ref, o_ref):
    z = jnp.maximum(
        p_ref[...].astype(jnp.float32)
        + parts_ref[0, 0:_N, :].astype(jnp.float32)
        + parts_ref[1, 0:_N, :].astype(jnp.float32)
        + b1_ref[...], 0.0)
    h = jnp.maximum(
        jnp.dot(z, w2_ref[...], preferred_element_type=jnp.float32)
        + b2_ref[...], 0.0)
    o_ref[...] = jnp.dot(h, w3_ref[...], preferred_element_type=jnp.float32
                         ).astype(jnp.bfloat16)


def _out_body(p_ref, parts_ref, b3_ref, w4_ref, b4_ref, o_ref):
    z = jnp.maximum(
        p_ref[...].astype(jnp.float32)
        + parts_ref[0, 0:_N, :].astype(jnp.float32)
        + parts_ref[1, 0:_N, :].astype(jnp.float32)
        + b3_ref[...], 0.0)
    o = jnp.dot(z, w4_ref[...], preferred_element_type=jnp.float32) + b4_ref[...]
    s = o - jnp.max(o, axis=1, keepdims=True)
    o_ref[...] = s - jnp.log(jnp.sum(jnp.exp(s), axis=1, keepdims=True))


_matmul = pl.pallas_call(
    _matmul_body,
    out_shape=jax.ShapeDtypeStruct((_N, _DH), jnp.bfloat16),
)

_mid = pl.pallas_call(
    _mid_body,
    out_shape=jax.ShapeDtypeStruct((_N, _DH), jnp.bfloat16),
)

_out = pl.pallas_call(
    _out_body,
    out_shape=jax.ShapeDtypeStruct((_N, _DOUT), jnp.float32),
)


def _make_segsum():
    mesh = plsc.VectorSubcoreMesh(core_axis_name="c", subcore_axis_name="s")

    @functools.partial(
        pl.kernel,
        mesh=mesh,
        compiler_params=pltpu.CompilerParams(use_tc_tiling_on_sc=False),
        out_type=jax.ShapeDtypeStruct((_NC, _NACC, _DH), jnp.bfloat16),
        scratch_types=[
            pltpu.VMEM((2, _K, _CHUNK), jnp.int32),
            pltpu.VMEM((_NB, _CHUNK, _DH), jnp.bfloat16),
            pltpu.VMEM_SHARED((_NACC, _DH), jnp.bfloat16),
        ] + [pltpu.SemaphoreType.DMA] * (2 * _NB),
    )
    def segsum(p_hbm, edge_hbm, out_hbm, idx_v, rows_v, acc, *sems):
        cid = lax.axis_index("c")
        sid = lax.axis_index("s")
        wid = cid * _NS + sid
        sem_g = sems[:_NB]
        sem_s = sems[_NB:]

        src_v = None
        dst_v = None

        def start_gather(j, b):
            pltpu.async_copy(p_hbm.at[src_v.at[j]], rows_v.at[b], sem_g[b])

        def wait_gather(j, b):
            pltpu.make_async_copy(
                p_hbm.at[src_v.at[j]], rows_v.at[b], sem_g[b]).wait()

        def start_scatter(j, b):
            pltpu.async_copy(rows_v.at[b], acc.at[dst_v.at[j]], sem_s[b],
                             add=True)

        def wait_scatter(j, b):
            pltpu.make_async_copy(
                rows_v.at[b], acc.at[dst_v.at[j]], sem_s[b]).wait()

        pltpu.sync_copy(edge_hbm.at[wid], idx_v)
        src_v = idx_v.at[0]
        dst_v = idx_v.at[1]

        for u in range(_LA):
            start_gather(u, u)

        def zrow(r, carry):
            for c in range(_DH // 32):
                rows_v[_NB - 1, r, pl.ds(c * 32, 32)] = jnp.zeros(
                    (32,), jnp.bfloat16)
            return carry
        lax.fori_loop(0, _ZR, zrow, 0)
        base = sid * _RPT
        for t in range(_RPT // _ZR):
            pltpu.sync_copy(rows_v.at[_NB - 1],
                            acc.at[pl.ds(base + t * _ZR, _ZR)])
        plsc.subcore_barrier()

        for u in range(_LA):
            wait_gather(u, u)
            start_scatter(u, u)
            start_gather(u + _LA, (u + _LA) % _NB)

        def body(it, carry):
            j0 = _LA + 8 * it
            for u in range(8):
                j = j0 + u
                b = (_LA + u) % _NB
                wait_gather(j, b)
                start_scatter(j, b)
                wait_scatter(j - _LA, u % _NB)
                start_gather(j + _LA, u % _NB)
            return carry

        lax.fori_loop(0, (_K - 2 * _LA) // 8, body, 0)

        for u in range(_LA):
            j = _K - _LA + u
            wait_gather(j, (_LA + u) % _NB)
            start_scatter(j, (_LA + u) % _NB)
            wait_scatter(j - _LA, u % _NB)
        for u in range(_LA):
            wait_scatter(_K - _LA + u, (_LA + u) % _NB)
        plsc.subcore_barrier()

        pltpu.sync_copy(acc.at[pl.ds(base, _RPT)],
                        out_hbm.at[cid, pl.ds(base, _RPT)])

    return segsum


_segsum = _make_segsum()


def kernel(x, edge_index, W1, b1, W2, b2, W3, b3, W4, b4):
    src = edge_index[0].astype(jnp.int32)
    dst = edge_index[1].astype(jnp.int32)
    npad = _EPAD - _E
    pad_src = jnp.arange(npad, dtype=jnp.int32) % _N
    pad_dst = _N + (jnp.arange(npad, dtype=jnp.int32) % (_NACC - _N))
    edges = jnp.stack([
        jnp.concatenate([src, pad_src]).reshape(_NW, _K, _CHUNK),
        jnp.concatenate([dst, pad_dst]).reshape(_NW, _K, _CHUNK),
    ], axis=1)

    b1r = b1.reshape(1, _DH)
    b2r = b2.reshape(1, _DH)
    b3r = b3.reshape(1, _DH)
    b4r = b4.reshape(1, _DOUT)

    p1 = _matmul(x, W1)
    parts1 = _segsum(p1, edges)
    p2 = _mid(p1, parts1, b1r, W2, b2r, W3)
    parts2 = _segsum(p2, edges)
    return _out(p2, parts2, b3r, W4, b4r)
